# Initial kernel scaffold; baseline (speedup 1.0000x reference)
import jax
import jax.numpy as jnp
from jax.experimental import pallas as pl


def kernel(events):
    raise NotImplementedError("write your pallas kernel here")


# speedup vs baseline: 67.3627x; 67.3627x over previous
import functools

import jax
import jax.numpy as jnp
import numpy as np
from jax import lax
from jax.experimental import pallas as pl
from jax.experimental.pallas import tpu as pltpu
from jax.experimental.pallas import tpu_sc as plsc

C, H, W = 9, 260, 346
NB = 8
NEV = 2_000_000
RB = NEV // NB
WH = W * H
WHC = WH * C
S = 2 * WHC
NC, NS = 2, 16
NCH = 123
GRP = 8
LROW = NS * NCH * 128
SPAD = 1_619_456
TSLICE = SPAD // NS
FL_LAST = S - (NS - 1) * TSLICE
ZCH = 4_096
NZF = TSLICE // ZCH
ZTAIL = TSLICE - NZF * ZCH
FTAIL_LAST = FL_LAST - NZF * ZCH

_BOUNDS = [np.float32(i / C) for i in range(1, C)]


def _idx_body(x_ref, y_ref, t_ref, p_ref, out_ref):
    t = t_ref[...]
    tn = t / jnp.max(t)
    bin_ = jnp.zeros(t.shape, jnp.int32)
    for cb in _BOUNDS:
        bin_ = bin_ + (tn > cb).astype(jnp.int32)
    xi = x_ref[...].astype(jnp.int32)
    yi = y_ref[...].astype(jnp.int32)
    pi = p_ref[...].astype(jnp.int32)
    idx = xi + W * yi + WH * bin_ + WHC * pi
    out_ref[...] = jnp.full((1, 1, LROW), S, jnp.int32)
    out_ref[:, :, :RB] = idx


def _compute_idx(xs, ys, ts, ps):
    return pl.pallas_call(
        _idx_body,
        grid=(NB,),
        in_specs=[pl.BlockSpec((1, 1, RB), lambda b: (b, 0, 0))] * 4,
        out_specs=pl.BlockSpec((1, 1, LROW), lambda b: (b, 0, 0)),
        out_shape=jax.ShapeDtypeStruct((NB, 1, LROW), jnp.int32),
    )(xs, ys, ts, ps)


@functools.cache
def _make_sc_hist():
    mesh = plsc.VectorSubcoreMesh(
        core_axis_name="c", subcore_axis_name="s", num_cores=NC, num_subcores=NS
    )

    @functools.partial(
        pl.kernel,
        out_type=jax.ShapeDtypeStruct((NB * S,), jnp.float32),
        mesh=mesh,
        scratch_types=[
            pltpu.VMEM((NCH, 128), jnp.int32),
            pltpu.VMEM((GRP, 128), jnp.float32),
            pltpu.VMEM((ZCH,), jnp.float32),
            pltpu.VMEM((ZCH,), jnp.float32),
            pltpu.VMEM_SHARED((SPAD,), jnp.float32),
            pltpu.SemaphoreType.DMA,
        ],
    )
    def _sc_hist(idx_hbm, ones_hbm, zeros_hbm, out_hbm, idx_v, ones_v,
                 zero_v, buf_v, hist, sem):
        c = lax.axis_index("c")
        s = lax.axis_index("s")
        pltpu.sync_copy(ones_hbm, ones_v)
        pltpu.sync_copy(zeros_hbm, zero_v)
        for bl in range(NB // NC):
            b = c * (NB // NC) + bl
            zoff = s * TSLICE
            zdescs = [
                pltpu.async_copy(zero_v,
                                 hist.at[pl.ds(zoff + k * ZCH, ZCH)], sem)
                for k in range(NZF)
            ]
            zdescs.append(
                pltpu.async_copy(zero_v.at[pl.ds(0, ZTAIL)],
                                 hist.at[pl.ds(zoff + NZF * ZCH, ZTAIL)],
                                 sem))
            for d in zdescs:
                d.wait()
            plsc.subcore_barrier()
            pltpu.sync_copy(idx_hbm.at[b, s], idx_v)

            def _grp(g, carry):
                descs = [
                    pltpu.async_copy(ones_v.at[j],
                                     hist.at[idx_v.at[g * GRP + j]],
                                     sem, add=True)
                    for j in range(GRP)
                ]
                for d in descs:
                    d.wait()
                return carry

            lax.fori_loop(0, NCH // GRP, _grp, 0)
            tail = [
                pltpu.async_copy(ones_v.at[j],
                                 hist.at[idx_v.at[(NCH // GRP) * GRP + j]],
                                 sem, add=True)
                for j in range(NCH % GRP)
            ]
            for d in tail:
                d.wait()
            plsc.subcore_barrier()
            ooff = b * S + s * TSLICE

            def _flush(k, carry):
                pltpu.sync_copy(hist.at[pl.ds(zoff + k * ZCH, ZCH)], buf_v)
                pltpu.sync_copy(buf_v, out_hbm.at[pl.ds(ooff + k * ZCH, ZCH)])
                return carry

            lax.fori_loop(0, NZF, _flush, 0)

            @pl.when(s < NS - 1)
            def _():
                pltpu.sync_copy(hist.at[pl.ds(zoff + NZF * ZCH, ZTAIL)],
                                buf_v.at[pl.ds(0, ZTAIL)])
                pltpu.sync_copy(buf_v.at[pl.ds(0, ZTAIL)],
                                out_hbm.at[pl.ds(ooff + NZF * ZCH, ZTAIL)])

            @pl.when(s == NS - 1)
            def _():
                pltpu.sync_copy(hist.at[pl.ds(zoff + NZF * ZCH, FTAIL_LAST)],
                                buf_v.at[pl.ds(0, FTAIL_LAST)])
                pltpu.sync_copy(buf_v.at[pl.ds(0, FTAIL_LAST)],
                                out_hbm.at[pl.ds(ooff + NZF * ZCH, FTAIL_LAST)])

    return _sc_hist


def kernel(events):
    xs = events[:, 0].reshape(NB, 1, RB)
    ys = events[:, 1].reshape(NB, 1, RB)
    ts = events[:, 2].reshape(NB, 1, RB)
    ps = events[:, 3].reshape(NB, 1, RB)
    idxp = _compute_idx(xs, ys, ts, ps)
    idx4 = idxp.reshape(NB, NS, NCH, 128)
    ones = jnp.ones((GRP, 128), jnp.float32)
    zeros = jnp.zeros((ZCH,), jnp.float32)
    vox = _make_sc_hist()(idx4, ones, zeros)
    return vox.reshape(NB, 2 * C, H, W)
